# Initial kernel scaffold; baseline (speedup 1.0000x reference)
import jax
import jax.numpy as jnp
from jax.experimental import pallas as pl


def kernel(x_1_true, lig_x, src_idxs, dst_idxs, lig_ue_mask, node_batch_idxs_lig, time_weights):
    raise NotImplementedError("write your pallas kernel here")


# speedup vs baseline: 188.3895x; 188.3895x over previous
import functools

import jax
import jax.numpy as jnp
from jax import lax
from jax.experimental import pallas as pl
from jax.experimental.pallas import tpu as pltpu
from jax.experimental.pallas import tpu_sc as plsc

N_NODES = 100000
N_EDGES = 6400000
D_MAX_SQ = 16.0
EPS = 1e-8

_info = plsc.get_sparse_core_info()
_NC = _info.num_cores
_NS = _info.num_subcores
_L = _info.num_lanes
_NW = _NC * _NS
_EPW = N_EDGES // _NW
_CHUNK = 2000
_NCHUNKS = _EPW // _CHUNK
_CV = _CHUNK // _L


def _rsqrt(x):
    k = plsc.bitcast(x, jnp.int32)
    r = plsc.bitcast(jnp.int32(0x5F3759DF) - (k >> 1), jnp.float32)
    for _ in range(3):
        r = r * (1.5 - 0.5 * x * r * r)
    return r


_mesh = plsc.VectorSubcoreMesh(core_axis_name="c", subcore_axis_name="s")


@functools.partial(
    pl.kernel,
    mesh=_mesh,
    compiler_params=pltpu.CompilerParams(
        needs_layout_passes=False, use_tc_tiling_on_sc=False),
    out_type=jax.ShapeDtypeStruct((_NW, 2, _L), jnp.float32),
    scratch_types=[
        pltpu.VMEM((_CHUNK,), jnp.int32),
        pltpu.VMEM((_CHUNK,), jnp.int32),
        pltpu.VMEM((_CHUNK,), jnp.float32),
        pltpu.VMEM((_CHUNK, 8), jnp.float32),
        pltpu.VMEM((_CHUNK, 8), jnp.float32),
        pltpu.VMEM((2, _L), jnp.float32),
        pltpu.SemaphoreType.DMA,
        pltpu.SemaphoreType.DMA,
    ],
)
def _edge_loss_sc(table, srci, dsti, maskf, out,
                  sidx_v, didx_v, m_v, srows_v, drows_v, acc_v, sem1, sem2):
    wid = lax.axis_index("s") * _NC + lax.axis_index("c")
    base0 = wid * _EPW
    iota = lax.iota(jnp.int32, _L)
    cols = [jnp.full((_L,), j, jnp.int32) for j in range(7)]

    def chunk_body(g, carry):
        num, cnt = carry
        base = base0 + g * _CHUNK
        pltpu.sync_copy(srci.at[pl.ds(base, _CHUNK)], sidx_v)
        pltpu.sync_copy(dsti.at[pl.ds(base, _CHUNK)], didx_v)
        pltpu.sync_copy(maskf.at[pl.ds(base, _CHUNK)], m_v)
        cp1 = pltpu.async_copy(table.at[sidx_v], srows_v, sem1)
        cp2 = pltpu.async_copy(table.at[didx_v], drows_v, sem2)
        cp1.wait()
        cp2.wait()

        def vec_body(i, carry2):
            num2, cnt2 = carry2
            row = i * _L + iota
            s = [plsc.load_gather(srows_v, [row, cols[j]]) for j in range(7)]
            d = [plsc.load_gather(drows_v, [row, cols[j]]) for j in range(6)]
            m = m_v[pl.ds(i * _L, _L)]
            t0 = s[0] - d[0]
            t1 = s[1] - d[1]
            t2 = s[2] - d[2]
            a = jnp.maximum(t0 * t0 + t1 * t1 + t2 * t2, EPS)
            g0 = s[3] - d[3]
            g1 = s[4] - d[4]
            g2 = s[5] - d[5]
            b = jnp.maximum(g0 * g0 + g1 * g1 + g2 * g2, EPS)
            ab = a * b
            sq = ab * _rsqrt(ab)
            sqerr = (a + b) - 2.0 * sq
            keep = jnp.where(a < D_MAX_SQ, m, 0.0)
            return (num2 + keep * s[6] * sqerr, cnt2 + keep)

        return lax.fori_loop(0, _CV, vec_body, (num, cnt))

    zero = jnp.zeros((_L,), jnp.float32)
    num, cnt = lax.fori_loop(0, _NCHUNKS, chunk_body, (zero, zero))
    acc_v[0, :] = num
    acc_v[1, :] = cnt
    pltpu.sync_copy(acc_v, out.at[wid])


def kernel(x_1_true, lig_x, src_idxs, dst_idxs, lig_ue_mask,
           node_batch_idxs_lig, time_weights):
    tw_node = time_weights[node_batch_idxs_lig]
    table = jnp.concatenate(
        [x_1_true, lig_x, tw_node[:, None],
         jnp.zeros((N_NODES, 1), jnp.float32)], axis=1)
    parts = _edge_loss_sc(table,
                          src_idxs.astype(jnp.int32),
                          dst_idxs.astype(jnp.int32),
                          lig_ue_mask.astype(jnp.float32))
    num = jnp.sum(parts[:, 0, :])
    cnt = jnp.sum(parts[:, 1, :])
    return num / jnp.maximum(cnt, 1.0)


# speedup vs baseline: 264.3731x; 1.4033x over previous
import functools

import jax
import jax.numpy as jnp
from jax import lax
from jax.experimental import pallas as pl
from jax.experimental.pallas import tpu as pltpu
from jax.experimental.pallas import tpu_sc as plsc

N_NODES = 100000
N_EDGES = 6400000
D_MAX_SQ = 16.0
EPS = 1e-8

_info = plsc.get_sparse_core_info()
_NC = _info.num_cores
_NS = _info.num_subcores
_L = _info.num_lanes
_NW = _NC * _NS
_EPW = N_EDGES // _NW
_CHUNK = 2000
_NCHUNKS = _EPW // _CHUNK
_NPAIRS = _NCHUNKS // 2
_CV = _CHUNK // _L


def _rsqrt(x):
    k = plsc.bitcast(x, jnp.int32)
    r = plsc.bitcast(jnp.int32(0x5F3759DF) - (k >> 1), jnp.float32)
    for _ in range(2):
        r = r * (1.5 - 0.5 * x * r * r)
    return r


_mesh = plsc.VectorSubcoreMesh(core_axis_name="c", subcore_axis_name="s")


@functools.partial(
    pl.kernel,
    mesh=_mesh,
    compiler_params=pltpu.CompilerParams(
        needs_layout_passes=False, use_tc_tiling_on_sc=False),
    out_type=jax.ShapeDtypeStruct((_NW, 2, _L), jnp.float32),
    scratch_types=[
        [pltpu.VMEM((_CHUNK,), jnp.int32)] * 2,
        [pltpu.VMEM((_CHUNK,), jnp.int32)] * 2,
        [pltpu.VMEM((_CHUNK,), jnp.float32)] * 2,
        [pltpu.VMEM((_CHUNK, 8), jnp.float32)] * 2,
        [pltpu.VMEM((_CHUNK, 8), jnp.float32)] * 2,
        pltpu.VMEM((2, _L), jnp.float32),
        [pltpu.SemaphoreType.DMA] * 2,
        [pltpu.SemaphoreType.DMA] * 2,
        [pltpu.SemaphoreType.DMA] * 2,
        [pltpu.SemaphoreType.DMA] * 2,
    ],
)
def _edge_loss_sc(table, srci, dsti, maskf, out,
                  sidx, didx, m, srows, drows, acc_v,
                  sem_li, sem_lm, sem_gs, sem_gd):
    wid = lax.axis_index("s") * _NC + lax.axis_index("c")
    base0 = wid * _EPW
    iota = lax.iota(jnp.int32, _L)
    cols = [jnp.full((_L,), j, jnp.int32) for j in range(7)]

    def lin_idx(g, p):
        base = base0 + g * _CHUNK
        return (pltpu.make_async_copy(srci.at[pl.ds(base, _CHUNK)],
                                      sidx[p], sem_li[p]),
                pltpu.make_async_copy(dsti.at[pl.ds(base, _CHUNK)],
                                      didx[p], sem_li[p]))

    def lin_mask(g, p):
        base = base0 + g * _CHUNK
        return pltpu.make_async_copy(maskf.at[pl.ds(base, _CHUNK)],
                                     m[p], sem_lm[p])

    def gathers(p):
        return (pltpu.make_async_copy(table.at[sidx[p]], srows[p], sem_gs[p]),
                pltpu.make_async_copy(table.at[didx[p]], drows[p], sem_gd[p]))

    def compute(p, num, cnt):
        sr, dr, mm = srows[p], drows[p], m[p]

        def body(i, carry):
            num2, cnt2 = carry
            row = i * _L + iota
            s = [plsc.load_gather(sr, [row, cols[j]]) for j in range(7)]
            d = [plsc.load_gather(dr, [row, cols[j]]) for j in range(6)]
            mv = mm[pl.ds(i * _L, _L)]
            t0 = s[0] - d[0]
            t1 = s[1] - d[1]
            t2 = s[2] - d[2]
            a = jnp.maximum(t0 * t0 + t1 * t1 + t2 * t2, EPS)
            g0 = s[3] - d[3]
            g1 = s[4] - d[4]
            g2 = s[5] - d[5]
            b = jnp.maximum(g0 * g0 + g1 * g1 + g2 * g2, EPS)
            ab = a * b
            sq = ab * _rsqrt(ab)
            sqerr = (a + b) - 2.0 * sq
            keep = jnp.where(a < D_MAX_SQ, mv, 0.0)
            return (num2 + keep * s[6] * sqerr, cnt2 + keep)

        return plsc.parallel_loop(0, _CV, 1, unroll=5,
                                  carry=(num, cnt))(body)

    for cp in lin_idx(0, 0) + lin_idx(1, 1):
        cp.start()
    lin_mask(0, 0).start()
    lin_mask(1, 1).start()
    for cp in lin_idx(0, 0):
        cp.wait()
    for cp in gathers(0):
        cp.start()

    def pair_body(g2, carry):
        num, cnt = carry
        for b in (0, 1):
            g = g2 * 2 + b
            p, q = b, 1 - b
            more = g2 < _NPAIRS - 1

            def stage_next():
                for cp in lin_idx(g + 1, q):
                    cp.wait()
                for cp in gathers(q):
                    cp.start()

            if b == 0:
                stage_next()
            else:
                pl.when(more)(stage_next)

            for cp in gathers(p):
                cp.wait()

            def prefetch_idx():
                for cp in lin_idx(g + 2, p):
                    cp.start()

            pl.when(more)(prefetch_idx)
            lin_mask(g, p).wait()
            num, cnt = compute(p, num, cnt)
            pl.when(more)(lambda: lin_mask(g + 2, p).start())
        return (num, cnt)

    zero = jnp.zeros((_L,), jnp.float32)
    num, cnt = lax.fori_loop(0, _NPAIRS, pair_body, (zero, zero))
    acc_v[0, :] = num
    acc_v[1, :] = cnt
    pltpu.sync_copy(acc_v, out.at[wid])


def kernel(x_1_true, lig_x, src_idxs, dst_idxs, lig_ue_mask,
           node_batch_idxs_lig, time_weights):
    tw_node = time_weights[node_batch_idxs_lig]
    table = jnp.concatenate(
        [x_1_true, lig_x, tw_node[:, None],
         jnp.zeros((N_NODES, 1), jnp.float32)], axis=1)
    parts = _edge_loss_sc(table,
                          src_idxs.astype(jnp.int32),
                          dst_idxs.astype(jnp.int32),
                          lig_ue_mask.astype(jnp.float32))
    num = jnp.sum(parts[:, 0, :])
    cnt = jnp.sum(parts[:, 1, :])
    return num / jnp.maximum(cnt, 1.0)


# speedup vs baseline: 428.6509x; 1.6214x over previous
import functools

import jax
import jax.numpy as jnp
from jax import lax
from jax.experimental import pallas as pl
from jax.experimental.pallas import tpu as pltpu
from jax.experimental.pallas import tpu_sc as plsc

N_NODES = 100000
N_EDGES = 6400000
D_MAX_SQ = 16.0
EPS = 1e-8

_info = plsc.get_sparse_core_info()
_NC = _info.num_cores
_NS = _info.num_subcores
_L = _info.num_lanes
_NW = _NC * _NS
_EPW = N_EDGES // _NW
_CHUNK = 2000
_NCHUNKS = _EPW // _CHUNK
_NPAIRS = _NCHUNKS // 2
_CV = _CHUNK // _L


def _rsqrt(x):
    k = plsc.bitcast(x, jnp.int32)
    r = plsc.bitcast(jnp.int32(0x5F3759DF) - (k >> 1), jnp.float32)
    for _ in range(2):
        r = r * (1.5 - 0.5 * x * r * r)
    return r


_mesh = plsc.VectorSubcoreMesh(core_axis_name="c", subcore_axis_name="s")


@functools.partial(
    pl.kernel,
    mesh=_mesh,
    compiler_params=pltpu.CompilerParams(
        needs_layout_passes=False, use_tc_tiling_on_sc=False),
    out_type=jax.ShapeDtypeStruct((_NW, 2, _L), jnp.float32),
    scratch_types=[
        [pltpu.VMEM((_CHUNK,), jnp.int32)] * 2,
        [pltpu.VMEM((_CHUNK,), jnp.int32)] * 2,
        [pltpu.VMEM((_CHUNK,), jnp.float32)] * 2,
        [pltpu.VMEM((_CHUNK, 8), jnp.float32)] * 2,
        [pltpu.VMEM((_CHUNK, 8), jnp.float32)] * 2,
        pltpu.VMEM((2, _L), jnp.float32),
        pltpu.VMEM_SHARED((N_NODES, 8), jnp.float32),
        [pltpu.SemaphoreType.DMA] * 2,
        [pltpu.SemaphoreType.DMA] * 2,
        [pltpu.SemaphoreType.DMA] * 2,
        [pltpu.SemaphoreType.DMA] * 2,
    ],
)
def _edge_loss_sc(table, srci, dsti, maskf, out,
                  sidx, didx, m, srows, drows, acc_v, spt,
                  sem_li, sem_lm, sem_gs, sem_gd):
    sid = lax.axis_index("s")
    wid = sid * _NC + lax.axis_index("c")
    base0 = wid * _EPW
    iota = lax.iota(jnp.int32, _L)
    cols = [jnp.full((_L,), j, jnp.int32) for j in range(7)]

    rpt = N_NODES // _NS
    pltpu.sync_copy(table.at[pl.ds(sid * rpt, rpt)],
                    spt.at[pl.ds(sid * rpt, rpt)])
    plsc.subcore_barrier()

    def lin_idx(g, p):
        base = base0 + g * _CHUNK
        return (pltpu.make_async_copy(srci.at[pl.ds(base, _CHUNK)],
                                      sidx[p], sem_li[p]),
                pltpu.make_async_copy(dsti.at[pl.ds(base, _CHUNK)],
                                      didx[p], sem_li[p]))

    def lin_mask(g, p):
        base = base0 + g * _CHUNK
        return pltpu.make_async_copy(maskf.at[pl.ds(base, _CHUNK)],
                                     m[p], sem_lm[p])

    def gathers(p):
        return (pltpu.make_async_copy(spt.at[sidx[p]], srows[p], sem_gs[p]),
                pltpu.make_async_copy(spt.at[didx[p]], drows[p], sem_gd[p]))

    def compute(p, num, cnt):
        sr, dr, mm = srows[p], drows[p], m[p]

        def body(i, carry):
            num2, cnt2 = carry
            row = i * _L + iota
            s = [plsc.load_gather(sr, [row, cols[j]]) for j in range(7)]
            d = [plsc.load_gather(dr, [row, cols[j]]) for j in range(6)]
            mv = mm[pl.ds(i * _L, _L)]
            t0 = s[0] - d[0]
            t1 = s[1] - d[1]
            t2 = s[2] - d[2]
            a = jnp.maximum(t0 * t0 + t1 * t1 + t2 * t2, EPS)
            g0 = s[3] - d[3]
            g1 = s[4] - d[4]
            g2 = s[5] - d[5]
            b = jnp.maximum(g0 * g0 + g1 * g1 + g2 * g2, EPS)
            ab = a * b
            sq = ab * _rsqrt(ab)
            sqerr = (a + b) - 2.0 * sq
            keep = jnp.where(a < D_MAX_SQ, mv, 0.0)
            return (num2 + keep * s[6] * sqerr, cnt2 + keep)

        return plsc.parallel_loop(0, _CV, 1, unroll=5,
                                  carry=(num, cnt))(body)

    for cp in lin_idx(0, 0) + lin_idx(1, 1):
        cp.start()
    lin_mask(0, 0).start()
    lin_mask(1, 1).start()
    for cp in lin_idx(0, 0):
        cp.wait()
    for cp in gathers(0):
        cp.start()

    def pair_body(g2, carry):
        num, cnt = carry
        for b in (0, 1):
            g = g2 * 2 + b
            p, q = b, 1 - b
            more = g2 < _NPAIRS - 1

            def stage_next():
                for cp in lin_idx(g + 1, q):
                    cp.wait()
                for cp in gathers(q):
                    cp.start()

            if b == 0:
                stage_next()
            else:
                pl.when(more)(stage_next)

            for cp in gathers(p):
                cp.wait()

            def prefetch_idx():
                for cp in lin_idx(g + 2, p):
                    cp.start()

            pl.when(more)(prefetch_idx)
            lin_mask(g, p).wait()
            num, cnt = compute(p, num, cnt)
            pl.when(more)(lambda: lin_mask(g + 2, p).start())
        return (num, cnt)

    zero = jnp.zeros((_L,), jnp.float32)
    num, cnt = lax.fori_loop(0, _NPAIRS, pair_body, (zero, zero))
    acc_v[0, :] = num
    acc_v[1, :] = cnt
    pltpu.sync_copy(acc_v, out.at[wid])


def kernel(x_1_true, lig_x, src_idxs, dst_idxs, lig_ue_mask,
           node_batch_idxs_lig, time_weights):
    tw_node = time_weights[node_batch_idxs_lig]
    table = jnp.concatenate(
        [x_1_true, lig_x, tw_node[:, None],
         jnp.zeros((N_NODES, 1), jnp.float32)], axis=1)
    parts = _edge_loss_sc(table,
                          src_idxs.astype(jnp.int32),
                          dst_idxs.astype(jnp.int32),
                          lig_ue_mask.astype(jnp.float32))
    num = jnp.sum(parts[:, 0, :])
    cnt = jnp.sum(parts[:, 1, :])
    return num / jnp.maximum(cnt, 1.0)
